# baseline (speedup 1.0000x reference)
import jax
import jax.numpy as jnp
from jax import lax
from jax.experimental import pallas as pl
from jax.experimental.pallas import tpu as pltpu

DEP = 128
SLOT = 128
GROUPS = 2
GROUP_SIZE = 1024
TOTAL = GROUPS * GROUP_SIZE
CHD = 8
NCH = DEP // CHD
NBUF = 4


def _body(addr_hbm, emb_ref, freq_ref, mem_ref, out_ref, abuf, fts, sem0, sem1, sem2, sem3):
    sems = (sem0, sem1, sem2, sem3)

    def copy(c, b):
        return pltpu.make_async_copy(
            addr_hbm.at[pl.ds(c * CHD, CHD)],
            abuf.at[pl.ds(b * CHD, CHD)],
            sems[b])

    copy(0, 0).start()
    copy(1, 1).start()
    copy(2, 2).start()
    fts[...] = emb_ref[...].T * freq_ref[...]

    def chunk(c, _):
        b = lax.rem(c, NBUF)
        for bb in range(NBUF):
            @pl.when(jnp.logical_and(b == bb, c + 3 < NCH))
            def _issue(bb=bb):
                copy(c + 3, (bb + 3) % NBUF).start()

            @pl.when(b == bb)
            def _wait(bb=bb):
                copy(0, bb).wait()

        a = abuf[pl.ds(b * CHD, CHD)]
        ftc = fts[pl.ds(c * CHD, CHD), :]
        c0 = jnp.sum(a[:, :GROUP_SIZE, :] * ftc[:, :GROUP_SIZE, None], axis=1)
        c1 = jnp.sum(a[:, GROUP_SIZE:, :] * ftc[:, GROUP_SIZE:, None], axis=1)
        out_ref[0, pl.ds(c * CHD, CHD), :] = mem_ref[0, pl.ds(c * CHD, CHD), :] + c0
        out_ref[1, pl.ds(c * CHD, CHD), :] = mem_ref[1, pl.ds(c * CHD, CHD), :] + c1
        return 0

    lax.fori_loop(0, NCH, chunk, 0)


def kernel(batch_address, batch_embedding, batch_frequency, memory_matrix):
    return pl.pallas_call(
        _body,
        in_specs=[
            pl.BlockSpec(memory_space=pltpu.MemorySpace.HBM),
            pl.BlockSpec((TOTAL, DEP), lambda: (0, 0)),
            pl.BlockSpec((1, TOTAL), lambda: (0, 0)),
            pl.BlockSpec((GROUPS, DEP, SLOT), lambda: (0, 0, 0)),
        ],
        out_specs=pl.BlockSpec((GROUPS, DEP, SLOT), lambda: (0, 0, 0)),
        out_shape=jax.ShapeDtypeStruct((GROUPS, DEP, SLOT), jnp.float32),
        scratch_shapes=[
            pltpu.VMEM((NBUF * CHD, TOTAL, SLOT), jnp.float32),
            pltpu.VMEM((DEP, TOTAL), jnp.float32),
            pltpu.SemaphoreType.DMA,
            pltpu.SemaphoreType.DMA,
            pltpu.SemaphoreType.DMA,
            pltpu.SemaphoreType.DMA,
        ],
        compiler_params=pltpu.CompilerParams(
            vmem_limit_bytes=100 * 1024 * 1024,
        ),
    )(batch_address, batch_embedding, batch_frequency[None, :], memory_matrix)


# speedup vs baseline: 1.0194x; 1.0194x over previous
import jax
import jax.numpy as jnp
from jax import lax
from jax.experimental import pallas as pl
from jax.experimental.pallas import tpu as pltpu

DEP = 128
SLOT = 128
GROUPS = 2
GROUP_SIZE = 1024
TOTAL = GROUPS * GROUP_SIZE
CHD = 8
NCH = DEP // CHD
NBUF = 3


def _body(addr_hbm, emb_ref, freq_ref, mem_ref, out_ref, abuf, fts, sem0, sem1, sem2):
    sems = (sem0, sem1, sem2)

    def copy(c, b):
        return pltpu.make_async_copy(
            addr_hbm.at[pl.ds(c * CHD, CHD)],
            abuf.at[pl.ds(b * CHD, CHD)],
            sems[b])

    copy(0, 0).start()
    copy(1, 1).start()
    fts[...] = emb_ref[...].T * freq_ref[...]

    def chunk(c, _):
        b = lax.rem(c, NBUF)
        for bb in range(NBUF):
            @pl.when(jnp.logical_and(b == bb, c + 2 < NCH))
            def _issue(bb=bb):
                copy(c + 2, (bb + 2) % NBUF).start()

            @pl.when(b == bb)
            def _wait(bb=bb):
                copy(0, bb).wait()

        a = abuf[pl.ds(b * CHD, CHD)]
        ftc = fts[pl.ds(c * CHD, CHD), :]
        c0 = jnp.sum(a[:, :GROUP_SIZE, :] * ftc[:, :GROUP_SIZE, None], axis=1)
        c1 = jnp.sum(a[:, GROUP_SIZE:, :] * ftc[:, GROUP_SIZE:, None], axis=1)
        out_ref[0, pl.ds(c * CHD, CHD), :] = mem_ref[0, pl.ds(c * CHD, CHD), :] + c0
        out_ref[1, pl.ds(c * CHD, CHD), :] = mem_ref[1, pl.ds(c * CHD, CHD), :] + c1
        return 0

    lax.fori_loop(0, NCH, chunk, 0)


def kernel(batch_address, batch_embedding, batch_frequency, memory_matrix):
    return pl.pallas_call(
        _body,
        in_specs=[
            pl.BlockSpec(memory_space=pltpu.MemorySpace.HBM),
            pl.BlockSpec((TOTAL, DEP), lambda: (0, 0)),
            pl.BlockSpec((1, TOTAL), lambda: (0, 0)),
            pl.BlockSpec((GROUPS, DEP, SLOT), lambda: (0, 0, 0)),
        ],
        out_specs=pl.BlockSpec((GROUPS, DEP, SLOT), lambda: (0, 0, 0)),
        out_shape=jax.ShapeDtypeStruct((GROUPS, DEP, SLOT), jnp.float32),
        scratch_shapes=[
            pltpu.VMEM((NBUF * CHD, TOTAL, SLOT), jnp.float32),
            pltpu.VMEM((DEP, TOTAL), jnp.float32),
            pltpu.SemaphoreType.DMA,
            pltpu.SemaphoreType.DMA,
            pltpu.SemaphoreType.DMA,
        ],
        compiler_params=pltpu.CompilerParams(
            vmem_limit_bytes=100 * 1024 * 1024,
        ),
    )(batch_address, batch_embedding, batch_frequency[None, :], memory_matrix)
